# Initial kernel scaffold; baseline (speedup 1.0000x reference)
import jax
import jax.numpy as jnp
from jax.experimental import pallas as pl


def kernel(user_emb, item_emb, edge_values, layer_u_w, layer_i_w, concat_u_w, concat_i_w, edges_u, edges_i):
    raise NotImplementedError("write your pallas kernel here")


# speedup vs baseline: 1.1616x; 1.1616x over previous
import functools

import jax
import jax.numpy as jnp
from jax import lax
from jax.experimental import pallas as pl
from jax.experimental.pallas import tpu as pltpu
from jax.experimental.pallas import tpu_sc as plsc

N_NODES = 50000
DIM = 64
HALF = 32
N_FB = 4
N_LAYERS = 2
N_EDGES = 1_000_000

NC = 2
NS = 16
CHUNK = 512
N_CHUNKS = 123
EPT = N_CHUNKS * CHUNK
N_PAD = NS * EPT

N_ACC = 50048
ROWS_PER_SUB = N_ACC // NS
ZROWS = 184

_mesh = plsc.VectorSubcoreMesh(core_axis_name="c", subcore_axis_name="s")


@functools.partial(
    pl.kernel,
    mesh=_mesh,
    compiler_params=pltpu.CompilerParams(
        use_tc_tiling_on_sc=False, needs_layout_passes=False),
    out_type=jax.ShapeDtypeStruct((NC * N_ACC, HALF), jnp.float32),
    scratch_types=[
        pltpu.VMEM((CHUNK,), jnp.int32),
        pltpu.VMEM((CHUNK,), jnp.int32),
        pltpu.VMEM((CHUNK,), jnp.float32),
        pltpu.VMEM((CHUNK, HALF), jnp.float32),
        pltpu.VMEM((ZROWS, HALF), jnp.float32),
        pltpu.VMEM_SHARED((N_ACC, HALF), jnp.float32),
        pltpu.SemaphoreType.DMA,
    ],
)
def _segsum_sc(table_hbm, src_hbm, dst_hbm, ev_hbm, out_hbm,
               src_v, dst_v, ev_v, rows_v, zero_v, acc, sem):
    cid = lax.axis_index("c")
    sid = lax.axis_index("s")

    zeros16 = jnp.zeros((16,), jnp.float32)

    def zfill(i, _):
        zero_v[i, pl.ds(0, 16)] = zeros16
        zero_v[i, pl.ds(16, 16)] = zeros16
        return 0

    lax.fori_loop(0, ZROWS, zfill, 0)

    def zcopy(i, _):
        pltpu.sync_copy(zero_v, acc.at[pl.ds(sid * ROWS_PER_SUB + i * ZROWS, ZROWS)])
        return 0

    lax.fori_loop(0, ROWS_PER_SUB // ZROWS, zcopy, 0)
    plsc.subcore_barrier()

    base0 = sid * EPT
    table_off = cid * N_NODES

    def chunk_body(ci, _):
        base = base0 + ci * CHUNK
        pltpu.sync_copy(src_hbm.at[pl.ds(base, CHUNK)], src_v)
        pltpu.sync_copy(dst_hbm.at[pl.ds(base, CHUNK)], dst_v)
        pltpu.sync_copy(ev_hbm.at[pl.ds(base, CHUNK)], ev_v)

        def adj(j, _):
            src_v[pl.ds(j * 16, 16)] = src_v[pl.ds(j * 16, 16)] + table_off
            return 0

        lax.fori_loop(0, CHUNK // 16, adj, 0)

        pltpu.async_copy(table_hbm.at[src_v], rows_v, sem).wait()

        iota16 = lax.iota(jnp.int32, 16)

        def scale(g, _):
            e0 = g * 16
            ev16 = ev_v[pl.ds(e0, 16)]
            rows = e0 + iota16
            for j in range(HALF):
                col = jnp.full((16,), j, jnp.int32)
                v = plsc.load_gather(rows_v, [rows, col])
                plsc.store_scatter(rows_v, [rows, col], v * ev16)
            return 0

        lax.fori_loop(0, CHUNK // 16, scale, 0)

        pltpu.sync_copy(rows_v, acc.at[dst_v], add=True)
        return 0

    lax.fori_loop(0, N_CHUNKS, chunk_body, 0)
    plsc.subcore_barrier()

    r0 = sid * ROWS_PER_SUB
    pltpu.sync_copy(acc.at[pl.ds(r0, ROWS_PER_SUB)],
                    out_hbm.at[pl.ds(cid * N_ACC + r0, ROWS_PER_SUB)])


_RB = 2000


def _layer_body(x_ref, w_ref, o_ref):
    a = jnp.concatenate([x_ref[0], x_ref[1]], axis=-1)
    y = jax.nn.sigmoid(jnp.dot(a, w_ref[...], preferred_element_type=jnp.float32))
    o_ref[0] = y[:, :HALF]
    o_ref[1] = y[:, HALF:]


_layer_tc = pl.pallas_call(
    _layer_body,
    grid=(N_NODES // _RB,),
    in_specs=[
        pl.BlockSpec((2, _RB, HALF), lambda i: (0, i, 0)),
        pl.BlockSpec((DIM, DIM), lambda i: (0, 0)),
    ],
    out_specs=pl.BlockSpec((2, _RB, HALF), lambda i: (0, i, 0)),
    out_shape=jax.ShapeDtypeStruct((2, N_NODES, HALF), jnp.float32),
)


def _concat_body(xa_ref, xb_ref, w_ref, o_ref):
    a = jnp.concatenate([xa_ref[0], xa_ref[1]], axis=-1)
    b = jnp.concatenate([xb_ref[0], xb_ref[1]], axis=-1)
    w = w_ref[...]
    o_ref[...] = (jnp.dot(a, w[:DIM], preferred_element_type=jnp.float32)
                  + jnp.dot(b, w[DIM:], preferred_element_type=jnp.float32))


_concat_tc = pl.pallas_call(
    _concat_body,
    grid=(N_NODES // _RB,),
    in_specs=[
        pl.BlockSpec((2, _RB, HALF), lambda i: (0, i, 0)),
        pl.BlockSpec((2, _RB, HALF), lambda i: (0, i, 0)),
        pl.BlockSpec((2 * DIM, DIM), lambda i: (0, 0)),
    ],
    out_specs=pl.BlockSpec((_RB, DIM), lambda i: (i, 0)),
    out_shape=jax.ShapeDtypeStruct((N_NODES, DIM), jnp.float32),
)


def _halves(x):
    return x.reshape(N_NODES, 2, HALF).transpose(1, 0, 2).reshape(NC * N_NODES, HALF)


def kernel(user_emb, item_emb, edge_values, layer_u_w, layer_i_w,
           concat_u_w, concat_i_w, edges_u, edges_i):
    pad = N_PAD - N_EDGES
    eu = jnp.pad(edges_u, ((0, 0), (0, pad)))
    ei = jnp.pad(edges_i, ((0, 0), (0, pad)))
    ev = jnp.pad(edge_values, ((0, 0), (0, pad)))

    uh = _halves(user_emb)
    ih = _halves(item_emb)

    flat = lambda x: x.reshape(NC * N_NODES, HALF)
    unflat = lambda x: x.reshape(2, N_ACC, HALF)[:, :N_NODES, :]

    user_outs = []
    item_outs = []
    for b in range(N_FB):
        agg_u = unflat(_segsum_sc(ih, ei[b], eu[b], ev[b]))
        agg_i = unflat(_segsum_sc(uh, eu[b], ei[b], ev[b]))
        u1 = _layer_tc(agg_u, layer_u_w[b, 0])
        i1 = _layer_tc(agg_i, layer_i_w[b, 0])
        agg_u2 = unflat(_segsum_sc(flat(i1), ei[b], eu[b], ev[b]))
        agg_i2 = unflat(_segsum_sc(flat(u1), eu[b], ei[b], ev[b]))
        u2 = _layer_tc(agg_u2, layer_u_w[b, 1])
        i2 = _layer_tc(agg_i2, layer_i_w[b, 1])
        user_outs.append(_concat_tc(u1, u2, concat_u_w[b]))
        item_outs.append(_concat_tc(i1, i2, concat_i_w[b]))

    return jnp.concatenate([jnp.stack(user_outs), jnp.stack(item_outs)], axis=1)
